# baseline (speedup 1.0000x reference)
import functools

import jax
import jax.numpy as jnp
from jax import lax
from jax.experimental import pallas as pl
from jax.experimental.pallas import tpu as pltpu
from jax.experimental.pallas import tpu_sc as plsc

_NUM_CHUNKS = 2
_BLK = 128
_SUB = 256
_NUM_WORKERS = 32


def _route_body(w_ref, p0_ref, p1_ref, w0_ref, w1_ref, s_ref, bc_ref):
    w = w_ref[...]
    ch, c = w.shape
    lane = lax.broadcasted_iota(jnp.int32, w.shape, 1)

    m0 = jnp.max(w, axis=1, keepdims=True)
    e0 = jnp.min(jnp.where(w == m0, lane, c), axis=1, keepdims=True)
    oh0 = lane == e0
    w_rest = jnp.where(oh0, -jnp.inf, w)
    m1 = jnp.max(w_rest, axis=1, keepdims=True)
    e1 = jnp.min(jnp.where(w_rest == m1, lane, c), axis=1, keepdims=True)
    oh1 = lane == e1
    denom = jnp.maximum(m0 + m1, 1e-8)
    w0_ref[...] = m0 / denom
    w1_ref[...] = m1 / denom
    s_ref[...] = (m0 + m1) / denom

    cnt = jnp.where(jnp.logical_or(oh0, oh1), 1.0, 0.0)
    ri = lax.broadcasted_iota(jnp.int32, (_SUB, _SUB), 0)
    ci = lax.broadcasted_iota(jnp.int32, (_SUB, _SUB), 1)
    ltri = jnp.where(ri > ci, 1.0, 0.0).astype(jnp.bfloat16)
    parts = []
    base = jnp.zeros((1, c), jnp.float32)
    for j in range(ch // _SUB):
        blk = cnt[j * _SUB:(j + 1) * _SUB]
        ploc = lax.dot_general(
            ltri, blk.astype(jnp.bfloat16),
            dimension_numbers=(((1,), (0,)), ((), ())),
            preferred_element_type=jnp.float32)
        parts.append(ploc + base)
        base = base + jnp.sum(blk, axis=0, keepdims=True)
    pfx = jnp.concatenate(parts, axis=0)
    tot = base

    nblk = jnp.floor((tot + (_BLK - 1)) / _BLK)
    r8 = lax.broadcasted_iota(jnp.int32, (c, c), 0)
    c8 = lax.broadcasted_iota(jnp.int32, (c, c), 1)
    ltri8 = jnp.where(r8 < c8, 1.0, 0.0).astype(jnp.bfloat16)
    cumb = lax.dot_general(
        nblk.astype(jnp.bfloat16), ltri8,
        dimension_numbers=(((1,), (0,)), ((), ())),
        preferred_element_type=jnp.float32)
    segstart = cumb * _BLK

    slot = pfx + segstart
    p0_ref[...] = jnp.sum(jnp.where(oh0, slot, 0.0), axis=1,
                          keepdims=True).astype(jnp.int32)
    p1_ref[...] = jnp.sum(jnp.where(oh1, slot, 0.0), axis=1,
                          keepdims=True).astype(jnp.int32)

    nb64 = bc_ref.shape[1]
    jj = lax.broadcasted_iota(jnp.int32, (c, nb64), 1).astype(jnp.float32)
    cumb_col = jnp.reshape(cumb, (c, 1))
    hits = jnp.where(cumb_col <= jj, 1, 0)
    bc_ref[...] = jnp.clip(jnp.sum(hits, axis=0, keepdims=True) - 1, 0, c - 1)


def _route(wc):
    ch, c = wc.shape
    return pl.pallas_call(
        _route_body,
        grid=(1,),
        in_specs=[pl.BlockSpec((ch, c), lambda i: (0, 0))],
        out_specs=[
            pl.BlockSpec((ch, 1), lambda i: (0, 0)),
            pl.BlockSpec((ch, 1), lambda i: (0, 0)),
            pl.BlockSpec((ch, 1), lambda i: (0, 0)),
            pl.BlockSpec((ch, 1), lambda i: (0, 0)),
            pl.BlockSpec((ch, 1), lambda i: (0, 0)),
            pl.BlockSpec((1, 64), lambda i: (0, 0)),
        ],
        out_shape=[
            jax.ShapeDtypeStruct((ch, 1), jnp.int32),
            jax.ShapeDtypeStruct((ch, 1), jnp.int32),
            jax.ShapeDtypeStruct((ch, 1), jnp.float32),
            jax.ShapeDtypeStruct((ch, 1), jnp.float32),
            jax.ShapeDtypeStruct((ch, 1), jnp.float32),
            jax.ShapeDtypeStruct((1, 64), jnp.int32),
        ],
    )(wc)


def _sc_scatter(qc, p0, p1, npad):
    ch, d = qc.shape
    tpw = ch // _NUM_WORKERS
    mesh = plsc.VectorSubcoreMesh(core_axis_name="c", subcore_axis_name="s")

    @functools.partial(
        pl.kernel, mesh=mesh,
        out_type=jax.ShapeDtypeStruct((npad, d), jnp.float32),
        scratch_types=[
            pltpu.VMEM((tpw,), jnp.int32),
            pltpu.VMEM((tpw, d), jnp.float32),
            pltpu.SemaphoreType.DMA,
        ],
    )
    def scatter_kernel(q_hbm, p0_hbm, p1_hbm, qg_hbm, idx_v, rows_v, sem):
        wid = lax.axis_index("s") * 2 + lax.axis_index("c")
        base = wid * tpw
        pltpu.sync_copy(q_hbm.at[pl.ds(base, tpw)], rows_v)
        pltpu.sync_copy(p0_hbm.at[pl.ds(base, tpw)], idx_v)
        pltpu.async_copy(rows_v, qg_hbm.at[idx_v], sem).wait()
        pltpu.sync_copy(p1_hbm.at[pl.ds(base, tpw)], idx_v)
        pltpu.async_copy(rows_v, qg_hbm.at[idx_v], sem).wait()

    return scatter_kernel(qc, p0, p1)


def _sc_gather(yg, p0, p1, ch):
    npad, d = yg.shape
    tpw = ch // _NUM_WORKERS
    mesh = plsc.VectorSubcoreMesh(core_axis_name="c", subcore_axis_name="s")

    @functools.partial(
        pl.kernel, mesh=mesh,
        out_type=(jax.ShapeDtypeStruct((ch, d), jnp.float32),
                  jax.ShapeDtypeStruct((ch, d), jnp.float32)),
        scratch_types=[
            pltpu.VMEM((tpw,), jnp.int32),
            pltpu.VMEM((tpw, d), jnp.float32),
            pltpu.SemaphoreType.DMA,
        ],
    )
    def gather_kernel(yg_hbm, p0_hbm, p1_hbm, y0_hbm, y1_hbm,
                      idx_v, rows_v, sem):
        wid = lax.axis_index("s") * 2 + lax.axis_index("c")
        base = wid * tpw
        pltpu.sync_copy(p0_hbm.at[pl.ds(base, tpw)], idx_v)
        pltpu.async_copy(yg_hbm.at[idx_v], rows_v, sem).wait()
        pltpu.sync_copy(rows_v, y0_hbm.at[pl.ds(base, tpw)])
        pltpu.sync_copy(p1_hbm.at[pl.ds(base, tpw)], idx_v)
        pltpu.async_copy(yg_hbm.at[idx_v], rows_v, sem).wait()
        pltpu.sync_copy(rows_v, y1_hbm.at[pl.ds(base, tpw)])

    return gather_kernel(yg, p0, p1)


def _gmm_body(bc_ref, qg_ref, e_ref, yg_ref):
    qb = qg_ref[...].astype(jnp.bfloat16)
    yg_ref[...] = lax.dot_general(
        qb, e_ref[0],
        dimension_numbers=(((1,), (1,)), ((), ())),
        preferred_element_type=jnp.float32)


def _gmm(bc, qg, resid):
    npad, d = qg.shape
    nblk = npad // _BLK
    grid_spec = pltpu.PrefetchScalarGridSpec(
        num_scalar_prefetch=1,
        grid=(nblk,),
        in_specs=[
            pl.BlockSpec((_BLK, d), lambda j, bc: (j, 0)),
            pl.BlockSpec((1, d, d), lambda j, bc: (bc[j], 0, 0)),
        ],
        out_specs=pl.BlockSpec((_BLK, d), lambda j, bc: (j, 0)),
    )
    return pl.pallas_call(
        _gmm_body,
        grid_spec=grid_spec,
        out_shape=jax.ShapeDtypeStruct((npad, d), jnp.float32),
        compiler_params=pltpu.CompilerParams(
            dimension_semantics=("arbitrary",),
        ),
    )(bc, qg, resid)


def _combine_body(q_ref, y0_ref, y1_ref, w0_ref, w1_ref, s_ref, o_ref):
    o_ref[...] = (w0_ref[...] * y0_ref[...] + w1_ref[...] * y1_ref[...]
                  - s_ref[...] * q_ref[...])


def _combine(qc, y0, y1, w0, w1, s):
    ch, d = qc.shape
    t = 1024
    return pl.pallas_call(
        _combine_body,
        grid=(ch // t,),
        in_specs=[
            pl.BlockSpec((t, d), lambda i: (i, 0)),
            pl.BlockSpec((t, d), lambda i: (i, 0)),
            pl.BlockSpec((t, d), lambda i: (i, 0)),
            pl.BlockSpec((t, 1), lambda i: (i, 0)),
            pl.BlockSpec((t, 1), lambda i: (i, 0)),
            pl.BlockSpec((t, 1), lambda i: (i, 0)),
        ],
        out_specs=pl.BlockSpec((t, d), lambda i: (i, 0)),
        out_shape=jax.ShapeDtypeStruct((ch, d), jnp.float32),
    )(qc, y0, y1, w0, w1, s)


@jax.jit
def kernel(q, weights, W_stack):
    b, d = q.shape
    c = W_stack.shape[0]
    ch = b // _NUM_CHUNKS
    npad = 2 * ch + c * _BLK
    nblk = npad // _BLK

    resid = (W_stack + jnp.eye(d, dtype=W_stack.dtype)).astype(jnp.bfloat16)

    outs = []
    for cidx in range(_NUM_CHUNKS):
        qc = q[cidx * ch:(cidx + 1) * ch]
        wc = weights[cidx * ch:(cidx + 1) * ch]
        p0, p1, w0, w1, s, bc = _route(wc)
        p0f = p0.reshape(ch)
        p1f = p1.reshape(ch)
        qg = _sc_scatter(qc, p0f, p1f, npad)
        yg = _gmm(bc.reshape(64)[:nblk], qg, resid)
        y0, y1 = _sc_gather(yg, p0f, p1f, ch)
        outs.append(_combine(qc, y0, y1, w0, w1, s))
    return jnp.concatenate(outs, axis=0)


# speedup vs baseline: 1.5882x; 1.5882x over previous
import jax
import jax.numpy as jnp
from jax import lax
from jax.experimental import pallas as pl
from jax.experimental.pallas import tpu as pltpu

_NUM_TB = 4


def _moe_body(w_ref, q_ref, wk_ref, o_ref, qb_ref, wmb_ref):
    k = pl.program_id(1)
    t, d = q_ref.shape
    c = w_ref.shape[1]

    @pl.when(k == 0)
    def _():
        w = w_ref[...]
        lane = lax.broadcasted_iota(jnp.int32, w.shape, 1)
        m0 = jnp.max(w, axis=1, keepdims=True)
        e0 = jnp.min(jnp.where(w == m0, lane, c), axis=1, keepdims=True)
        oh0 = lane == e0
        w_rest = jnp.where(oh0, -jnp.inf, w)
        m1 = jnp.max(w_rest, axis=1, keepdims=True)
        e1 = jnp.min(jnp.where(w_rest == m1, lane, c), axis=1, keepdims=True)
        oh1 = lane == e1
        denom = jnp.maximum(m0 + m1, 1e-8)
        wm = (jnp.where(oh0, m0, 0.0) + jnp.where(oh1, m1, 0.0)) / denom
        s = (m0 + m1) / denom

        qf = q_ref[...]
        qb_ref[...] = qf.astype(jnp.bfloat16)
        o_ref[...] = (-s) * qf
        for kk in range(c):
            wmb_ref[kk] = jnp.broadcast_to(wm[:, kk:kk + 1], (t, 128))

    wb = wk_ref[0]
    ri = lax.broadcasted_iota(jnp.int32, wb.shape, 0)
    ci = lax.broadcasted_iota(jnp.int32, wb.shape, 1)
    eb = (wb + jnp.where(ri == ci, 1.0, 0.0)).astype(jnp.bfloat16)

    y = lax.dot_general(
        qb_ref[...], eb,
        dimension_numbers=(((1,), (1,)), ((), ())),
        preferred_element_type=jnp.float32,
    )
    wcol = wmb_ref[k]
    o_ref[...] += (y.reshape(t, d // 128, 128) * wcol[:, None, :]).reshape(t, d)


@jax.jit
def kernel(q, weights, W_stack):
    b, d = q.shape
    c = W_stack.shape[0]
    t = b // _NUM_TB

    return pl.pallas_call(
        _moe_body,
        grid=(_NUM_TB, c),
        in_specs=[
            pl.BlockSpec((t, c), lambda tb, k: (tb, 0)),
            pl.BlockSpec((t, d), lambda tb, k: (tb, 0)),
            pl.BlockSpec((1, d, d), lambda tb, k: (k, 0, 0)),
        ],
        out_specs=pl.BlockSpec((t, d), lambda tb, k: (tb, 0)),
        out_shape=jax.ShapeDtypeStruct((b, d), jnp.float32),
        scratch_shapes=[
            pltpu.VMEM((b // _NUM_TB, d), jnp.bfloat16),
            pltpu.VMEM((c, b // _NUM_TB, 128), jnp.float32),
        ],
        compiler_params=pltpu.CompilerParams(
            dimension_semantics=("arbitrary", "arbitrary"),
        ),
    )(weights, q, W_stack)


# speedup vs baseline: 2.2612x; 1.4238x over previous
import jax
import jax.numpy as jnp
from jax.experimental import pallas as pl
from jax.experimental.pallas import tpu as pltpu

_TOKEN_BLOCK = 256


def _moe_body(w_ref, q_ref, r_ref, o_ref):
    w = w_ref[...]
    num_charts = w.shape[1]
    lane = jax.lax.broadcasted_iota(jnp.int32, w.shape, 1)

    m0 = jnp.max(w, axis=1, keepdims=True)
    e0 = jnp.min(jnp.where(w == m0, lane, num_charts), axis=1, keepdims=True)
    oh0 = lane == e0
    w_rest = jnp.where(oh0, -jnp.inf, w)
    m1 = jnp.max(w_rest, axis=1, keepdims=True)
    e1 = jnp.min(jnp.where(w_rest == m1, lane, num_charts), axis=1, keepdims=True)
    oh1 = lane == e1

    denom = jnp.maximum(m0 + m1, 1e-8)
    wm = (jnp.where(oh0, m0, 0.0) + jnp.where(oh1, m1, 0.0)) / denom
    s = (m0 + m1) / denom

    qf = q_ref[...]
    qb = qf.astype(jnp.bfloat16)
    ys = [
        jax.lax.dot_general(
            qb, r_ref[k],
            dimension_numbers=(((1,), (1,)), ((), ())),
            preferred_element_type=jnp.float32,
        )
        for k in range(num_charts)
    ]
    acc = (-s) * qf
    for k in range(num_charts):
        acc = acc + wm[:, k:k + 1] * ys[k]
    o_ref[...] = acc


@jax.jit
def kernel(q, weights, W_stack):
    b, d = q.shape
    c = W_stack.shape[0]
    resid_t = (W_stack + jnp.eye(d, dtype=W_stack.dtype)).astype(jnp.bfloat16)

    t = _TOKEN_BLOCK
    grid = (b // t,)
    return pl.pallas_call(
        _moe_body,
        grid=grid,
        in_specs=[
            pl.BlockSpec((t, c), lambda i: (i, 0)),
            pl.BlockSpec((t, d), lambda i: (i, 0)),
            pl.BlockSpec((c, d, d), lambda i: (0, 0, 0)),
        ],
        out_specs=pl.BlockSpec((t, d), lambda i: (i, 0)),
        out_shape=jax.ShapeDtypeStruct((b, d), jnp.float32),
        compiler_params=pltpu.CompilerParams(
            dimension_semantics=("parallel",),
        ),
    )(weights, q, resid_t)


# speedup vs baseline: 2.3252x; 1.0283x over previous
import jax
import jax.numpy as jnp
from jax.experimental import pallas as pl
from jax.experimental.pallas import tpu as pltpu

_TOKEN_BLOCK = 1024


def _moe_body(w_ref, q_ref, r_ref, o_ref):
    w = w_ref[...]
    num_charts = w.shape[1]
    lane = jax.lax.broadcasted_iota(jnp.int32, w.shape, 1)

    m0 = jnp.max(w, axis=1, keepdims=True)
    e0 = jnp.min(jnp.where(w == m0, lane, num_charts), axis=1, keepdims=True)
    oh0 = lane == e0
    w_rest = jnp.where(oh0, -jnp.inf, w)
    m1 = jnp.max(w_rest, axis=1, keepdims=True)
    e1 = jnp.min(jnp.where(w_rest == m1, lane, num_charts), axis=1, keepdims=True)
    oh1 = lane == e1

    denom = jnp.maximum(m0 + m1, 1e-8)
    wm = (jnp.where(oh0, m0, 0.0) + jnp.where(oh1, m1, 0.0)) / denom
    s = (m0 + m1) / denom

    qf = q_ref[...]
    qb = qf.astype(jnp.bfloat16)
    ys = [
        jax.lax.dot_general(
            qb, r_ref[k],
            dimension_numbers=(((1,), (1,)), ((), ())),
            preferred_element_type=jnp.float32,
        )
        for k in range(num_charts)
    ]
    acc = (-s) * qf
    for k in range(num_charts):
        acc = acc + wm[:, k:k + 1] * ys[k]
    o_ref[...] = acc


@jax.jit
def kernel(q, weights, W_stack):
    b, d = q.shape
    c = W_stack.shape[0]
    resid_t = (W_stack + jnp.eye(d, dtype=W_stack.dtype)).astype(jnp.bfloat16)

    t = _TOKEN_BLOCK
    grid = (b // t,)
    return pl.pallas_call(
        _moe_body,
        grid=grid,
        in_specs=[
            pl.BlockSpec((t, c), lambda i: (i, 0)),
            pl.BlockSpec((t, d), lambda i: (i, 0)),
            pl.BlockSpec((c, d, d), lambda i: (0, 0, 0)),
        ],
        out_specs=pl.BlockSpec((t, d), lambda i: (i, 0)),
        out_shape=jax.ShapeDtypeStruct((b, d), jnp.float32),
        compiler_params=pltpu.CompilerParams(
            dimension_semantics=("parallel",),
        ),
    )(weights, q, resid_t)
